# baseline (speedup 1.0000x reference)
import functools
import math

import jax
import jax.numpy as jnp
from jax import lax
from jax.experimental import pallas as pl
from jax.experimental.pallas import tpu as pltpu
from jax.experimental.pallas import tpu_sc as plsc

D_MODEL = 64
SCALE = math.sqrt(D_MODEL)

_info = plsc.get_sparse_core_info()
NC = _info.num_cores
NS = _info.num_subcores
L = _info.num_lanes
NW = NC * NS

B = 4096
S = 200
IDX_W = 128
TI = B // IDX_W
N_UNITS = S * TI
UPW = N_UNITS // NW
VPR = D_MODEL // L
NBUF = 4
LOOK = 2


V = 1000000
VT = V // IDX_W
VT_REM = V - VT * IDX_W


def _make_relayout_kernel():
  mesh = plsc.VectorSubcoreMesh(core_axis_name="c", subcore_axis_name="s")

  @functools.partial(
      pl.kernel,
      mesh=mesh,
      out_type=jax.ShapeDtypeStruct((V // 2, 2 * D_MODEL), jnp.float32),
      scratch_types=[
          pltpu.VMEM((D_MODEL, IDX_W), jnp.float32),
          pltpu.VMEM((D_MODEL, 130), jnp.float32),
          pltpu.VMEM((VT_REM // 2, 2 * D_MODEL), jnp.float32),
          pltpu.SemaphoreType.DMA,
          pltpu.SemaphoreType.DMA,
      ],
      compiler_params=pltpu.CompilerParams(use_tc_tiling_on_sc=True,
                                           needs_layout_passes=False),
  )
  def relayout_kernel(tt_hbm, tail_hbm, out_hbm, sbuf, vbuf, tailbuf,
                      isem, osem):
    wid = lax.axis_index("s") * NC + lax.axis_index("c")
    base = wid * (VT // NW) + jnp.minimum(wid, 5)
    count = (VT // NW) + jnp.where(wid < 5, 1, 0)

    def do_unit(v0):
      v0 = pl.multiple_of(v0, IDX_W)
      copies = []
      for td in range(8):
        copies.append(pltpu.async_copy(
            tt_hbm.at[pl.ds(td * 8, 8), pl.ds(v0, IDX_W)],
            sbuf.at[pl.ds(td * 8, 8)], isem))
      for c in copies:
        c.wait()

      @plsc.parallel_loop(0, D_MODEL, step=1, unroll=2)
      def _(d):
        dv = jnp.zeros((L,), jnp.int32) + d
        for lb in range(IDX_W // L):
          lvec = lax.iota(jnp.int32, L) + (lb * L)
          v = sbuf[d, pl.ds(lb * L, L)]
          plsc.store_scatter(
              vbuf, [lvec // 2, lax.rem(lvec, 2) * D_MODEL + dv], v)

      pltpu.async_copy(
          vbuf.at[pl.ds(0, D_MODEL), pl.ds(0, 2 * D_MODEL)],
          out_hbm.at[pl.ds(pl.multiple_of(v0 // 2, D_MODEL), D_MODEL)],
          osem).wait()

    def body(i, carry):
      do_unit((base + i) * IDX_W)
      return carry

    lax.fori_loop(0, count, body, 0)
    @pl.when(wid == NW - 1)
    def _():
      pltpu.sync_copy(tail_hbm, tailbuf)
      pltpu.sync_copy(tailbuf, out_hbm.at[pl.ds(VT * D_MODEL, VT_REM // 2)])

  return relayout_kernel


def _make_kernel():
  mesh = plsc.VectorSubcoreMesh(core_axis_name="c", subcore_axis_name="s")

  @functools.partial(
      pl.kernel,
      mesh=mesh,
      out_type=jax.ShapeDtypeStruct((S, 8, TI, 8, IDX_W), jnp.float32),
      scratch_types=[
          pltpu.VMEM((UPW, IDX_W), jnp.int32),
          pltpu.VMEM((IDX_W, D_MODEL), jnp.float32),
          pltpu.VMEM((IDX_W, D_MODEL), jnp.float32),
          pltpu.VMEM((IDX_W, D_MODEL), jnp.float32),
          pltpu.VMEM((IDX_W, D_MODEL), jnp.float32),
          pltpu.VMEM((8, 8, IDX_W + 1), jnp.float32),
          pltpu.VMEM((8, 8, IDX_W + 1), jnp.float32),
          pltpu.VMEM((8, 8, IDX_W + 1), jnp.float32),
          pltpu.VMEM((8, 8, IDX_W + 1), jnp.float32),
          pltpu.SemaphoreType.DMA,
          pltpu.SemaphoreType.DMA,
          pltpu.SemaphoreType.DMA,
          pltpu.SemaphoreType.DMA,
          pltpu.SemaphoreType.DMA,
          pltpu.SemaphoreType.DMA,
          pltpu.SemaphoreType.DMA,
          pltpu.SemaphoreType.DMA,
      ],
      compiler_params=pltpu.CompilerParams(use_tc_tiling_on_sc=False,
                                           needs_layout_passes=False),
  )
  def emb_kernel(xt_hbm, table_hbm, out_hbm, idx_v, b0, b1, b2, b3,
                 t0, t1, t2, t3, g0, g1, g2, g3, o0, o1, o2, o3):
    bufs = (b0, b1, b2, b3)
    tbufs = (t0, t1, t2, t3)
    gsems = (g0, g1, g2, g3)
    osems = (o0, o1, o2, o3)
    wid = lax.axis_index("s") * NC + lax.axis_index("c")
    ubase = wid * UPW
    pltpu.sync_copy(xt_hbm.at[pl.ds(ubase, UPW)], idx_v)

    def unit_j_ti(u):
      return (u // (8 * TI)) * 8 + lax.rem(u, 8), lax.rem(u // 8, TI)

    def fire_gather(lu, buf, sem):
      return pltpu.async_copy(table_hbm.at[idx_v.at[lu]], buf, sem)

    def drain_gather(buf, sem):
      pltpu.make_async_copy(table_hbm.at[idx_v.at[0]], buf, sem).wait()

    def transpose_scale(buf, tbuf):
      @plsc.parallel_loop(0, IDX_W, step=1, unroll=2)
      def _(r):
        rv = jnp.zeros((L,), jnp.int32) + r
        for c in range(VPR):
          dvec = lax.iota(jnp.int32, L) + (c * L)
          v = buf[r, pl.ds(c * L, L)]
          plsc.store_scatter(tbuf, [dvec // 8, lax.rem(dvec, 8), rv],
                             v * SCALE)

    def fire_out(lu, tbuf, sem):
      j, ti = unit_j_ti(ubase + lu)
      return pltpu.async_copy(
          tbuf.at[pl.ds(0, 8), pl.ds(0, 8), pl.ds(0, IDX_W)],
          out_hbm.at[j, pl.ds(0, 8), ti, pl.ds(0, 8)], sem)

    def drain_out(tbuf, sem):
      pltpu.make_async_copy(
          tbuf.at[pl.ds(0, 8), pl.ds(0, 8), pl.ds(0, IDX_W)],
          out_hbm.at[0, pl.ds(0, 8), 0, pl.ds(0, 8)], sem).wait()

    for p in range(LOOK):
      fire_gather(p, bufs[p], gsems[p])

    def body(i, carry):
      for p in range(NBUF):
        lu = i * NBUF + p
        pn = (p + LOOK) % NBUF
        if p + LOOK < NBUF:
          fire_gather(lu + LOOK, bufs[pn], gsems[pn])
        else:
          @pl.when(i < UPW // NBUF - 1)
          def _():
            fire_gather(lu + LOOK, bufs[pn], gsems[pn])
        drain_gather(bufs[p], gsems[p])
        @pl.when(i > 0)
        def _():
          drain_out(tbufs[p], osems[p])
        transpose_scale(bufs[p], tbufs[p])
        fire_out(lu, tbufs[p], osems[p])
      return carry

    lax.fori_loop(0, UPW // NBUF, body, 0)
    for p in range(NBUF):
      drain_out(tbufs[p], osems[p])

  return emb_kernel


_emb_kernel = _make_kernel()
_relayout_kernel = _make_relayout_kernel()


@jax.jit
def kernel(x, table):
  xt = (
      x.astype(jnp.int32)
      .T.reshape(25, 8, TI, IDX_W)
      .transpose(0, 2, 1, 3)
      .reshape(N_UNITS, IDX_W)
  )
  tail = table[VT * IDX_W:].reshape(VT_REM // 2, 2 * D_MODEL)
  tlin = _relayout_kernel(table.T, tail).reshape(V, D_MODEL)
  out = _emb_kernel(xt, tlin)
  return out.transpose(2, 4, 0, 1, 3).reshape(B, S, D_MODEL)


# speedup vs baseline: 1.3272x; 1.3272x over previous
import functools
import math

import jax
import jax.numpy as jnp
from jax import lax
from jax.experimental import pallas as pl
from jax.experimental.pallas import tpu as pltpu
from jax.experimental.pallas import tpu_sc as plsc

D_MODEL = 64
SCALE = math.sqrt(D_MODEL)

_info = plsc.get_sparse_core_info()
NC = _info.num_cores
NS = _info.num_subcores
L = _info.num_lanes
NW = NC * NS

B = 4096
S = 200
IDX_W = 128
TI = B // IDX_W
N_UNITS = S * TI
UPW = N_UNITS // NW
VPR = D_MODEL // L
NBUF = 4
LOOK = 2


V = 1000000
VT = V // IDX_W
VT_REM = V - VT * IDX_W


def _make_relayout_kernel():
  mesh = plsc.VectorSubcoreMesh(core_axis_name="c", subcore_axis_name="s")

  @functools.partial(
      pl.kernel,
      mesh=mesh,
      out_type=jax.ShapeDtypeStruct((V // 2, 2 * D_MODEL), jnp.float32),
      scratch_types=[
          pltpu.VMEM((D_MODEL, IDX_W), jnp.float32),
          pltpu.VMEM((D_MODEL, IDX_W), jnp.float32),
          pltpu.VMEM((D_MODEL, 130), jnp.float32),
          pltpu.VMEM((D_MODEL, 130), jnp.float32),
          pltpu.VMEM((VT_REM // 2, 2 * D_MODEL), jnp.float32),
          pltpu.SemaphoreType.DMA,
          pltpu.SemaphoreType.DMA,
          pltpu.SemaphoreType.DMA,
          pltpu.SemaphoreType.DMA,
      ],
      compiler_params=pltpu.CompilerParams(use_tc_tiling_on_sc=True,
                                           needs_layout_passes=False),
  )
  def relayout_kernel(tt_hbm, tail_hbm, out_hbm, s0, s1, v0b, v1b, tailbuf,
                      i0, i1, o0, o1):
    sbufs, vbufs, isems, osems = (s0, s1), (v0b, v1b), (i0, i1), (o0, o1)
    wid = lax.axis_index("s") * NC + lax.axis_index("c")
    UPW1 = 246
    base = jnp.minimum(wid * UPW1, VT - UPW1)

    def fire_windows(u, sbuf, sem):
      v0 = pl.multiple_of(u * IDX_W, IDX_W)
      for td in range(8):
        pltpu.async_copy(
            tt_hbm.at[pl.ds(td * 8, 8), pl.ds(v0, IDX_W)],
            sbuf.at[pl.ds(td * 8, 8)], sem)

    def drain_windows(sbuf, sem):
      for td in range(8):
        pltpu.make_async_copy(
            tt_hbm.at[pl.ds(0, 8), pl.ds(0, IDX_W)],
            sbuf.at[pl.ds(td * 8, 8)], sem).wait()

    def transpose(sbuf, vbuf):
      @plsc.parallel_loop(0, D_MODEL, step=1, unroll=4)
      def _(d):
        dv = jnp.zeros((L,), jnp.int32) + d
        for lb in range(IDX_W // L):
          lvec = lax.iota(jnp.int32, L) + (lb * L)
          v = sbuf[d, pl.ds(lb * L, L)]
          plsc.store_scatter(
              vbuf, [lvec // 2, lax.rem(lvec, 2) * D_MODEL + dv], v)

    def fire_out(u, vbuf, sem):
      r0 = pl.multiple_of(u * D_MODEL, D_MODEL)
      pltpu.async_copy(
          vbuf.at[pl.ds(0, D_MODEL), pl.ds(0, 2 * D_MODEL)],
          out_hbm.at[pl.ds(r0, D_MODEL)], sem)

    def drain_out(vbuf, sem):
      pltpu.make_async_copy(
          vbuf.at[pl.ds(0, D_MODEL), pl.ds(0, 2 * D_MODEL)],
          out_hbm.at[pl.ds(0, D_MODEL)], sem).wait()

    fire_windows(base, sbufs[0], isems[0])

    def body(i, carry):
      for p in range(2):
        u = base + 2 * i + p
        pn = 1 - p
        if p == 0:
          fire_windows(u + 1, sbufs[pn], isems[pn])
        else:
          @pl.when(i < UPW1 // 2 - 1)
          def _():
            fire_windows(u + 1, sbufs[pn], isems[pn])
        drain_windows(sbufs[p], isems[p])
        @pl.when(i > 0)
        def _():
          drain_out(vbufs[p], osems[p])
        transpose(sbufs[p], vbufs[p])
        fire_out(u, vbufs[p], osems[p])
      return carry

    lax.fori_loop(0, UPW1 // 2, body, 0)
    for p in range(2):
      drain_out(vbufs[p], osems[p])
    @pl.when(wid == NW - 1)
    def _():
      pltpu.sync_copy(tail_hbm, tailbuf)
      pltpu.sync_copy(tailbuf, out_hbm.at[pl.ds(VT * D_MODEL, VT_REM // 2)])

  return relayout_kernel


def _make_kernel():
  mesh = plsc.VectorSubcoreMesh(core_axis_name="c", subcore_axis_name="s")

  @functools.partial(
      pl.kernel,
      mesh=mesh,
      out_type=jax.ShapeDtypeStruct((S, 8, TI, 8, IDX_W), jnp.float32),
      scratch_types=[
          pltpu.VMEM((UPW, IDX_W), jnp.int32),
          pltpu.VMEM((IDX_W, D_MODEL), jnp.float32),
          pltpu.VMEM((IDX_W, D_MODEL), jnp.float32),
          pltpu.VMEM((IDX_W, D_MODEL), jnp.float32),
          pltpu.VMEM((IDX_W, D_MODEL), jnp.float32),
          pltpu.VMEM((8, 8, IDX_W + 1), jnp.float32),
          pltpu.VMEM((8, 8, IDX_W + 1), jnp.float32),
          pltpu.VMEM((8, 8, IDX_W + 1), jnp.float32),
          pltpu.VMEM((8, 8, IDX_W + 1), jnp.float32),
          pltpu.SemaphoreType.DMA,
          pltpu.SemaphoreType.DMA,
          pltpu.SemaphoreType.DMA,
          pltpu.SemaphoreType.DMA,
          pltpu.SemaphoreType.DMA,
          pltpu.SemaphoreType.DMA,
          pltpu.SemaphoreType.DMA,
          pltpu.SemaphoreType.DMA,
      ],
      compiler_params=pltpu.CompilerParams(use_tc_tiling_on_sc=False,
                                           needs_layout_passes=False),
  )
  def emb_kernel(xt_hbm, table_hbm, out_hbm, idx_v, b0, b1, b2, b3,
                 t0, t1, t2, t3, g0, g1, g2, g3, o0, o1, o2, o3):
    bufs = (b0, b1, b2, b3)
    tbufs = (t0, t1, t2, t3)
    gsems = (g0, g1, g2, g3)
    osems = (o0, o1, o2, o3)
    wid = lax.axis_index("s") * NC + lax.axis_index("c")
    ubase = wid * UPW
    pltpu.sync_copy(xt_hbm.at[pl.ds(ubase, UPW)], idx_v)

    def unit_j_ti(u):
      return (u // (8 * TI)) * 8 + lax.rem(u, 8), lax.rem(u // 8, TI)

    def fire_gather(lu, buf, sem):
      return pltpu.async_copy(table_hbm.at[idx_v.at[lu]], buf, sem)

    def drain_gather(buf, sem):
      pltpu.make_async_copy(table_hbm.at[idx_v.at[0]], buf, sem).wait()

    def transpose_scale(buf, tbuf):
      @plsc.parallel_loop(0, IDX_W, step=1, unroll=2)
      def _(r):
        rv = jnp.zeros((L,), jnp.int32) + r
        for c in range(VPR):
          dvec = lax.iota(jnp.int32, L) + (c * L)
          v = buf[r, pl.ds(c * L, L)]
          plsc.store_scatter(tbuf, [dvec // 8, lax.rem(dvec, 8), rv],
                             v * SCALE)

    def fire_out(lu, tbuf, sem):
      j, ti = unit_j_ti(ubase + lu)
      return pltpu.async_copy(
          tbuf.at[pl.ds(0, 8), pl.ds(0, 8), pl.ds(0, IDX_W)],
          out_hbm.at[j, pl.ds(0, 8), ti, pl.ds(0, 8)], sem)

    def drain_out(tbuf, sem):
      pltpu.make_async_copy(
          tbuf.at[pl.ds(0, 8), pl.ds(0, 8), pl.ds(0, IDX_W)],
          out_hbm.at[0, pl.ds(0, 8), 0, pl.ds(0, 8)], sem).wait()

    for p in range(LOOK):
      fire_gather(p, bufs[p], gsems[p])

    def body(i, carry):
      for p in range(NBUF):
        lu = i * NBUF + p
        pn = (p + LOOK) % NBUF
        if p + LOOK < NBUF:
          fire_gather(lu + LOOK, bufs[pn], gsems[pn])
        else:
          @pl.when(i < UPW // NBUF - 1)
          def _():
            fire_gather(lu + LOOK, bufs[pn], gsems[pn])
        drain_gather(bufs[p], gsems[p])
        @pl.when(i > 0)
        def _():
          drain_out(tbufs[p], osems[p])
        transpose_scale(bufs[p], tbufs[p])
        fire_out(lu, tbufs[p], osems[p])
      return carry

    lax.fori_loop(0, UPW // NBUF, body, 0)
    for p in range(NBUF):
      drain_out(tbufs[p], osems[p])

  return emb_kernel


_emb_kernel = _make_kernel()
_relayout_kernel = _make_relayout_kernel()


@jax.jit
def kernel(x, table):
  xt = (
      x.astype(jnp.int32)
      .T.reshape(25, 8, TI, IDX_W)
      .transpose(0, 2, 1, 3)
      .reshape(N_UNITS, IDX_W)
  )
  tail = table[VT * IDX_W:].reshape(VT_REM // 2, 2 * D_MODEL)
  tlin = _relayout_kernel(table.T, tail).reshape(V, D_MODEL)
  out = _emb_kernel(xt, tlin)
  return out.transpose(2, 4, 0, 1, 3).reshape(B, S, D_MODEL)


# speedup vs baseline: 1.8239x; 1.3743x over previous
import functools
import math

import jax
import jax.numpy as jnp
from jax import lax
from jax.experimental import pallas as pl
from jax.experimental.pallas import tpu as pltpu
from jax.experimental.pallas import tpu_sc as plsc

D_MODEL = 64
SCALE = math.sqrt(D_MODEL)

_info = plsc.get_sparse_core_info()
NC = _info.num_cores
NS = _info.num_subcores
L = _info.num_lanes
NW = NC * NS

B = 4096
S = 200
IDX_W = 128
TI = B // IDX_W
N_UNITS = S * TI
UPW = N_UNITS // NW
VPR = D_MODEL // L
NBUF = 4
LOOK = 2


def _make_kernel():
  mesh = plsc.VectorSubcoreMesh(core_axis_name="c", subcore_axis_name="s")

  @functools.partial(
      pl.kernel,
      mesh=mesh,
      out_type=jax.ShapeDtypeStruct((S, 8, TI, 8, IDX_W), jnp.float32),
      scratch_types=[
          pltpu.VMEM((UPW, IDX_W), jnp.int32),
          pltpu.VMEM((IDX_W, D_MODEL), jnp.float32),
          pltpu.VMEM((IDX_W, D_MODEL), jnp.float32),
          pltpu.VMEM((IDX_W, D_MODEL), jnp.float32),
          pltpu.VMEM((IDX_W, D_MODEL), jnp.float32),
          pltpu.VMEM((8, 8, IDX_W + 1), jnp.float32),
          pltpu.VMEM((8, 8, IDX_W + 1), jnp.float32),
          pltpu.VMEM((8, 8, IDX_W + 1), jnp.float32),
          pltpu.VMEM((8, 8, IDX_W + 1), jnp.float32),
          pltpu.SemaphoreType.DMA,
          pltpu.SemaphoreType.DMA,
          pltpu.SemaphoreType.DMA,
          pltpu.SemaphoreType.DMA,
          pltpu.SemaphoreType.DMA,
          pltpu.SemaphoreType.DMA,
          pltpu.SemaphoreType.DMA,
          pltpu.SemaphoreType.DMA,
      ],
      compiler_params=pltpu.CompilerParams(use_tc_tiling_on_sc=False,
                                           needs_layout_passes=False),
  )
  def emb_kernel(xt_hbm, table_hbm, out_hbm, idx_v, b0, b1, b2, b3,
                 t0, t1, t2, t3, g0, g1, g2, g3, o0, o1, o2, o3):
    bufs = (b0, b1, b2, b3)
    tbufs = (t0, t1, t2, t3)
    gsems = (g0, g1, g2, g3)
    osems = (o0, o1, o2, o3)
    wid = lax.axis_index("s") * NC + lax.axis_index("c")
    ubase = wid * UPW
    pltpu.sync_copy(xt_hbm.at[pl.ds(ubase, UPW)], idx_v)

    def unit_j_ti(u):
      return (u // (8 * TI)) * 8 + lax.rem(u, 8), lax.rem(u // 8, TI)

    def fire_gather(lu, buf, sem):
      return pltpu.async_copy(table_hbm.at[idx_v.at[lu]], buf, sem)

    def drain_gather(buf, sem):
      pltpu.make_async_copy(table_hbm.at[idx_v.at[0]], buf, sem).wait()

    def transpose_scale(buf, tbuf):
      @plsc.parallel_loop(0, IDX_W, step=1, unroll=2)
      def _(r):
        rv = jnp.zeros((L,), jnp.int32) + r
        for c in range(VPR):
          dvec = lax.iota(jnp.int32, L) + (c * L)
          v = buf[r, pl.ds(c * L, L)]
          plsc.store_scatter(tbuf, [dvec // 8, lax.rem(dvec, 8), rv],
                             v * SCALE)

    def fire_out(lu, tbuf, sem):
      j, ti = unit_j_ti(ubase + lu)
      return pltpu.async_copy(
          tbuf.at[pl.ds(0, 8), pl.ds(0, 8), pl.ds(0, IDX_W)],
          out_hbm.at[j, pl.ds(0, 8), ti, pl.ds(0, 8)], sem)

    def drain_out(tbuf, sem):
      pltpu.make_async_copy(
          tbuf.at[pl.ds(0, 8), pl.ds(0, 8), pl.ds(0, IDX_W)],
          out_hbm.at[0, pl.ds(0, 8), 0, pl.ds(0, 8)], sem).wait()

    for p in range(LOOK):
      fire_gather(p, bufs[p], gsems[p])

    def body(i, carry):
      for p in range(NBUF):
        lu = i * NBUF + p
        pn = (p + LOOK) % NBUF
        if p + LOOK < NBUF:
          fire_gather(lu + LOOK, bufs[pn], gsems[pn])
        else:
          @pl.when(i < UPW // NBUF - 1)
          def _():
            fire_gather(lu + LOOK, bufs[pn], gsems[pn])
        drain_gather(bufs[p], gsems[p])
        @pl.when(i > 0)
        def _():
          drain_out(tbufs[p], osems[p])
        transpose_scale(bufs[p], tbufs[p])
        fire_out(lu, tbufs[p], osems[p])
      return carry

    lax.fori_loop(0, UPW // NBUF, body, 0)
    for p in range(NBUF):
      drain_out(tbufs[p], osems[p])

  return emb_kernel


_emb_kernel = _make_kernel()


@jax.jit
def kernel(x, table):
  xt = (
      x.astype(jnp.int32)
      .T.reshape(25, 8, TI, IDX_W)
      .transpose(0, 2, 1, 3)
      .reshape(N_UNITS, IDX_W)
  )
  out = _emb_kernel(xt, table)
  return out.transpose(2, 4, 0, 1, 3).reshape(B, S, D_MODEL)
